# Initial kernel scaffold; baseline (speedup 1.0000x reference)
import jax
import jax.numpy as jnp
from jax.experimental import pallas as pl


def kernel(x, edge_index, edge_attr, W1, b1, gamma, beta, W2, b2):
    raise NotImplementedError("write your pallas kernel here")


# speedup vs baseline: 3.1296x; 3.1296x over previous
import jax
import jax.numpy as jnp
from jax import lax
from jax.experimental import pallas as pl
from jax.experimental.pallas import tpu as pltpu
from jax.experimental.pallas import tpu_sc as plsc

N = 10000
E = 320000
D = 128
DE = 16

NC = 2
NS = 16
NW = NC * NS
EPW = E // NW
CHUNK = 80
NCHUNK = EPW // CHUNK
NP_ = 10240
RPS = NP_ // NS
ZROWS = 128

_HI = lax.Precision.HIGHEST


def _nodeproj_body(x_ref, wa_ref, wb_ref, a_ref, b_ref):
    xb = x_ref[...]
    dn = (((1,), (1,)), ((), ()))
    a_ref[...] = lax.dot_general(xb, wa_ref[...], dn, precision=_HI)
    b_ref[...] = lax.dot_general(xb, wb_ref[...], dn, precision=_HI)


def _node_proj(x, wa, wb):
    BN = 1000
    return pl.pallas_call(
        _nodeproj_body,
        grid=(N // BN,),
        in_specs=[
            pl.BlockSpec((BN, D), lambda i: (i, 0)),
            pl.BlockSpec((D, D), lambda i: (0, 0)),
            pl.BlockSpec((D, D), lambda i: (0, 0)),
        ],
        out_specs=[
            pl.BlockSpec((BN, D), lambda i: (i, 0)),
            pl.BlockSpec((BN, D), lambda i: (i, 0)),
        ],
        out_shape=[
            jax.ShapeDtypeStruct((N, D), jnp.float32),
            jax.ShapeDtypeStruct((N, D), jnp.float32),
        ],
    )(x, wa, wb)


def _gather_body(a_hbm, b_hbm, ii_hbm, jj_hbm, g_hbm, idx_i, idx_j, av, bv, sem):
    c = lax.axis_index("c")
    s = lax.axis_index("s")
    base = (s * NC + c) * EPW

    def chunk(t, carry):
        off = base + t * CHUNK
        pltpu.sync_copy(ii_hbm.at[pl.ds(off, CHUNK)], idx_i)
        pltpu.sync_copy(jj_hbm.at[pl.ds(off, CHUNK)], idx_j)
        d1 = pltpu.async_copy(a_hbm.at[idx_i], av, sem)
        d2 = pltpu.async_copy(b_hbm.at[idx_j], bv, sem)
        d1.wait()
        d2.wait()

        def row(r, carry2):
            for q in range(D // 16):
                sl = pl.ds(q * 16, 16)
                av[r, sl] = av[r, sl] + bv[r, sl]
            return carry2

        lax.fori_loop(0, CHUNK, row, 0)
        pltpu.sync_copy(av, g_hbm.at[pl.ds(off, CHUNK)])
        return carry

    lax.fori_loop(0, NCHUNK, chunk, 0)


def _edge_gather(a, b, ii, jj):
    return pl.kernel(
        _gather_body,
        out_type=jax.ShapeDtypeStruct((E, D), jnp.float32),
        mesh=plsc.VectorSubcoreMesh(core_axis_name="c", subcore_axis_name="s"),
        scratch_types=[
            pltpu.VMEM((CHUNK,), jnp.int32),
            pltpu.VMEM((CHUNK,), jnp.int32),
            pltpu.VMEM((CHUNK, D), jnp.float32),
            pltpu.VMEM((CHUNK, D), jnp.float32),
            pltpu.SemaphoreType.DMA,
        ],
    )(a, b, ii, jj)


def _edgemlp_body(g_ref, ea_ref, wc_ref, b1_ref, gam_ref, bet_ref, h_ref):
    dn = (((1,), (1,)), ((), ()))
    h = g_ref[...] + lax.dot_general(ea_ref[...], wc_ref[...], dn, precision=_HI)
    h = h + b1_ref[...]
    mean = jnp.mean(h, axis=-1, keepdims=True)
    cent = h - mean
    var = jnp.mean(cent * cent, axis=-1, keepdims=True)
    h = cent * lax.rsqrt(var + 1e-5) * gam_ref[...] + bet_ref[...]
    h_ref[...] = h * 0.5 * (1.0 + lax.erf(h * 0.7071067811865476))


def _edge_mlp(g, ea, wc, b1r, gamr, betr):
    BE = 2000
    return pl.pallas_call(
        _edgemlp_body,
        grid=(E // BE,),
        in_specs=[
            pl.BlockSpec((BE, D), lambda i: (i, 0)),
            pl.BlockSpec((BE, DE), lambda i: (i, 0)),
            pl.BlockSpec((D, DE), lambda i: (0, 0)),
            pl.BlockSpec((1, D), lambda i: (0, 0)),
            pl.BlockSpec((1, D), lambda i: (0, 0)),
            pl.BlockSpec((1, D), lambda i: (0, 0)),
        ],
        out_specs=pl.BlockSpec((BE, D), lambda i: (i, 0)),
        out_shape=jax.ShapeDtypeStruct((E, D), jnp.float32),
    )(g, ea, wc, b1r, gamr, betr)


def _scatter_body(h_hbm, jj_hbm, accp_hbm, idx_j, rows, acc_sh, sem):
    c = lax.axis_index("c")
    s = lax.axis_index("s")
    base = (s * NC + c) * EPW
    zero16 = jnp.zeros((16,), jnp.float32)

    def zfill(r, carry):
        for q in range(D // 16):
            rows[r, pl.ds(q * 16, 16)] = zero16
        return carry

    lax.fori_loop(0, CHUNK, zfill, 0)

    def zcopy(q, carry):
        pltpu.sync_copy(rows, acc_sh.at[pl.ds(s * RPS + q * CHUNK, CHUNK)])
        return carry

    lax.fori_loop(0, RPS // CHUNK, zcopy, 0)
    plsc.subcore_barrier()

    def chunk(t, carry):
        off = base + t * CHUNK
        pltpu.sync_copy(jj_hbm.at[pl.ds(off, CHUNK)], idx_j)
        pltpu.sync_copy(h_hbm.at[pl.ds(off, CHUNK)], rows)
        pltpu.sync_copy(rows, acc_sh.at[idx_j], add=True)
        return carry

    lax.fori_loop(0, NCHUNK, chunk, 0)
    plsc.subcore_barrier()

    pltpu.sync_copy(acc_sh.at[pl.ds(s * RPS, RPS)],
                    accp_hbm.at[c, pl.ds(s * RPS, RPS)])


def _edge_scatter(h, jj):
    return pl.kernel(
        _scatter_body,
        out_type=jax.ShapeDtypeStruct((NC, NP_, D), jnp.float32),
        mesh=plsc.VectorSubcoreMesh(core_axis_name="c", subcore_axis_name="s"),
        scratch_types=[
            pltpu.VMEM((CHUNK,), jnp.int32),
            pltpu.VMEM((CHUNK, D), jnp.float32),
            pltpu.VMEM_SHARED((NP_, D), jnp.float32),
            pltpu.SemaphoreType.DMA,
        ],
    )(h, jj)


def _degb2_body(b2_hbm, jj_hbm, dbp_hbm, idx_j, brow, rows, db_sh, sem):
    c = lax.axis_index("c")
    s = lax.axis_index("s")
    base = (s * NC + c) * EPW
    zero16 = jnp.zeros((16,), jnp.float32)

    def zfill(r, carry):
        for q in range(D // 16):
            rows[r, pl.ds(q * 16, 16)] = zero16
        return carry

    lax.fori_loop(0, CHUNK, zfill, 0)

    def zcopy(q, carry):
        pltpu.sync_copy(rows, db_sh.at[pl.ds(s * RPS + q * CHUNK, CHUNK)])
        return carry

    lax.fori_loop(0, RPS // CHUNK, zcopy, 0)

    pltpu.sync_copy(b2_hbm, brow)

    def bfill(r, carry):
        for q in range(D // 16):
            sl = pl.ds(q * 16, 16)
            rows[r, sl] = brow[0, sl]
        return carry

    lax.fori_loop(0, CHUNK, bfill, 0)
    plsc.subcore_barrier()

    def chunk(t, carry):
        off = base + t * CHUNK
        pltpu.sync_copy(jj_hbm.at[pl.ds(off, CHUNK)], idx_j)
        pltpu.sync_copy(rows, db_sh.at[idx_j], add=True)
        return carry

    lax.fori_loop(0, NCHUNK, chunk, 0)
    plsc.subcore_barrier()

    pltpu.sync_copy(db_sh.at[pl.ds(s * RPS, RPS)],
                    dbp_hbm.at[c, pl.ds(s * RPS, RPS)])


def _deg_b2(b2r, jj):
    return pl.kernel(
        _degb2_body,
        out_type=jax.ShapeDtypeStruct((NC, NP_, D), jnp.float32),
        mesh=plsc.VectorSubcoreMesh(core_axis_name="c", subcore_axis_name="s"),
        scratch_types=[
            pltpu.VMEM((CHUNK,), jnp.int32),
            pltpu.VMEM((1, D), jnp.float32),
            pltpu.VMEM((CHUNK, D), jnp.float32),
            pltpu.VMEM_SHARED((NP_, D), jnp.float32),
            pltpu.SemaphoreType.DMA,
        ],
    )(b2r, jj)


def _out_body(acc_ref, db_ref, w2_ref, o_ref):
    p = acc_ref[0] + acc_ref[1]
    dn = (((1,), (1,)), ((), ()))
    o = lax.dot_general(p, w2_ref[...], dn, precision=_HI)
    o_ref[...] = o + db_ref[0] + db_ref[1]


def _node_out(accp, dbp, w2):
    BN = 1000
    return pl.pallas_call(
        _out_body,
        grid=(N // BN,),
        in_specs=[
            pl.BlockSpec((NC, BN, D), lambda i: (0, i, 0)),
            pl.BlockSpec((NC, BN, D), lambda i: (0, i, 0)),
            pl.BlockSpec((D, D), lambda i: (0, 0)),
        ],
        out_specs=pl.BlockSpec((BN, D), lambda i: (i, 0)),
        out_shape=jax.ShapeDtypeStruct((N, D), jnp.float32),
    )(accp, dbp, w2)


def kernel(x, edge_index, edge_attr, W1, b1, gamma, beta, W2, b2):
    ii = edge_index[0]
    jj = edge_index[1]
    wa = W1[:, :D]
    wb = W1[:, D:2 * D]
    wc = W1[:, 2 * D:]
    a, b = _node_proj(x, wa, wb)
    g = _edge_gather(a, b, ii, jj)
    h = _edge_mlp(g, edge_attr, wc, b1.reshape(1, D), gamma.reshape(1, D),
                  beta.reshape(1, D))
    accp = _edge_scatter(h, jj)
    dbp = _deg_b2(b2.reshape(1, D), jj)
    return _node_out(accp, dbp, W2)
